# Initial kernel scaffold; baseline (speedup 1.0000x reference)
import jax
import jax.numpy as jnp
from jax.experimental import pallas as pl


def kernel(vols, strikes, expiries, strike, expiry):
    raise NotImplementedError("write your pallas kernel here")


# speedup vs baseline: 2.8050x; 2.8050x over previous
import functools

import jax
import jax.numpy as jnp
from jax import lax
from jax.experimental import pallas as pl
from jax.experimental.pallas import tpu as pltpu
from jax.experimental.pallas import tpu_sc as plsc

L = 16
N_EXP = 1024
N_STR = 4096


def _count_le(grid_v, n, strides, x_vec, iota):
    base = iota * 0
    cnt = base
    for st in strides:
        probe = base + iota * st
        in_range = probe < n
        probe_c = jnp.minimum(probe, n - 1)
        vals = plsc.load_gather(grid_v, [probe_c])
        le = jnp.logical_and(vals <= x_vec, in_range)
        cnt = plsc.all_reduce_population_count(le)
        if st != 1:
            base = base + jnp.maximum(cnt - 1, 0) * st
    return base + cnt


def _body(vols_hbm, strikes_hbm, expiries_hbm, strike_hbm, expiry_hbm,
          out_hbm, strikes_v, expiries_v, strike_v, expiry_v, rows_v, out_v):
    c = lax.axis_index("c")
    s = lax.axis_index("s")

    @pl.when(jnp.logical_and(c == 0, s == 0))
    def _():
        pltpu.sync_copy(expiries_hbm, expiries_v)
        pltpu.sync_copy(expiry_hbm, expiry_v)
        pltpu.sync_copy(strikes_hbm, strikes_v)
        pltpu.sync_copy(strike_hbm, strike_v)

        iota = lax.iota(jnp.int32, L)
        zeros = iota * 0
        ones = zeros + 1

        t = expiry_v[...]
        n_e = _count_le(expiries_v, N_EXP, [64, 4, 1], t, iota)
        jhi = jnp.clip(n_e, 1, N_EXP - 1)
        jlo = jhi - 1
        jlo_s = jnp.max(jlo)
        pltpu.sync_copy(vols_hbm.at[pl.ds(jlo_s * N_STR, 2 * N_STR)], rows_v)
        e_lo = plsc.load_gather(expiries_v, [jlo])
        e_hi = plsc.load_gather(expiries_v, [jhi])
        u = jnp.clip((t - e_lo) / (e_hi - e_lo), 0.0, 1.0)

        s_first = plsc.load_gather(strikes_v, [zeros])
        s_last = plsc.load_gather(strikes_v, [zeros + (N_STR - 1)])
        k = jnp.clip(strike_v[...], s_first, s_last)
        n_s = _count_le(strikes_v, N_STR, [256, 16, 1], k, iota)
        idx = jnp.clip(n_s - 1, 0, N_STR - 2)
        k_lo = plsc.load_gather(strikes_v, [idx])
        k_hi = plsc.load_gather(strikes_v, [idx + 1])
        w = jnp.clip((k - k_lo) / (k_hi - k_lo), 0.0, 1.0)

        v00 = plsc.load_gather(rows_v, [idx])
        v01 = plsc.load_gather(rows_v, [idx + 1])
        v10 = plsc.load_gather(rows_v, [idx + N_STR])
        v11 = plsc.load_gather(rows_v, [idx + N_STR + 1])
        a0 = v00 + w * (v01 - v00)
        a1 = v10 + w * (v11 - v10)
        out_v[...] = a0 + u * (a1 - a0)
        pltpu.sync_copy(out_v, out_hbm)


@functools.partial(jax.jit, static_argnums=())
def kernel(vols, strikes, expiries, strike, expiry):
    strike_v = jnp.full((L,), strike, dtype=jnp.float32)
    expiry_v = jnp.full((L,), expiry, dtype=jnp.float32)
    mesh = plsc.VectorSubcoreMesh(core_axis_name="c", subcore_axis_name="s")
    run = pl.kernel(
        _body,
        out_type=jax.ShapeDtypeStruct((L,), jnp.float32),
        mesh=mesh,
        compiler_params=pltpu.CompilerParams(needs_layout_passes=False),
        scratch_types=[
            pltpu.VMEM((N_STR,), jnp.float32),
            pltpu.VMEM((N_EXP,), jnp.float32),
            pltpu.VMEM((L,), jnp.float32),
            pltpu.VMEM((L,), jnp.float32),
            pltpu.VMEM((2 * N_STR,), jnp.float32),
            pltpu.VMEM((L,), jnp.float32),
        ],
    )
    out = run(vols.reshape(-1), strikes, expiries, strike_v, expiry_v)
    return out[0]


# speedup vs baseline: 5.2857x; 1.8844x over previous
import jax
import jax.numpy as jnp
from jax import lax
from jax.experimental import pallas as pl
from jax.experimental.pallas import tpu as pltpu
from jax.experimental.pallas import tpu_sc as plsc

L = 16
N_EXP = 1024
N_STR = 4096
ROWS = 16
COLS = 256


def _count_le(grid_v, n, strides, x_vec, iota):
    base = iota * 0
    cnt = base
    for st in strides:
        probe = base + iota * st
        in_range = probe < n
        probe_c = jnp.minimum(probe, n - 1)
        vals = plsc.load_gather(grid_v, [probe_c])
        le = jnp.logical_and(vals <= x_vec, in_range)
        cnt = plsc.all_reduce_population_count(le)
        if st != 1:
            base = base + jnp.maximum(cnt - 1, 0) * st
    return base + cnt


def _body(vols_hbm, strikes_hbm, expiries_hbm, strike_hbm, expiry_hbm,
          out_hbm, strikes_v, expiries_v, strike_v, expiry_v, win_v, out_v,
          sem_e, sem_s, sem_w):
    c = lax.axis_index("c")
    s = lax.axis_index("s")

    @pl.when(jnp.logical_and(c == 0, s == 0))
    def _():
        cp_e = pltpu.async_copy(expiries_hbm, expiries_v, sem_e)
        cp_t = pltpu.async_copy(expiry_hbm, expiry_v, sem_e)
        cp_s = pltpu.async_copy(strikes_hbm, strikes_v, sem_s)
        cp_k = pltpu.async_copy(strike_hbm, strike_v, sem_s)

        iota = lax.iota(jnp.int32, L)
        zeros = iota * 0

        cp_e.wait()
        cp_t.wait()
        t = expiry_v[...]
        n_e = _count_le(expiries_v, N_EXP, [64, 4, 1], t, iota)
        jhi = jnp.clip(n_e, 1, N_EXP - 1)
        jlo = jhi - 1
        e_lo = plsc.load_gather(expiries_v, [jlo])
        e_hi = plsc.load_gather(expiries_v, [jhi])
        u = jnp.clip((t - e_lo) / (e_hi - e_lo), 0.0, 1.0)

        cp_s.wait()
        cp_k.wait()
        s_first = plsc.load_gather(strikes_v, [zeros])
        s_last = plsc.load_gather(strikes_v, [zeros + (N_STR - 1)])
        k = jnp.clip(strike_v[...], s_first, s_last)
        n_s = _count_le(strikes_v, N_STR, [256, 16, 1], k, iota)
        idx = jnp.clip(n_s - 1, 0, N_STR - 2)

        rbase = (jlo >> 3) << 3
        cbase = jnp.minimum((idx >> 7) << 7, N_STR - COLS)
        rbase_s = pl.multiple_of(jnp.max(rbase), 8)
        cbase_s = pl.multiple_of(jnp.max(cbase), 128)
        cp_w = pltpu.async_copy(
            vols_hbm.at[pl.ds(rbase_s, ROWS), pl.ds(cbase_s, COLS)],
            win_v, sem_w)

        k_lo = plsc.load_gather(strikes_v, [idx])
        k_hi = plsc.load_gather(strikes_v, [idx + 1])
        w = jnp.clip((k - k_lo) / (k_hi - k_lo), 0.0, 1.0)

        r0 = jlo - rbase
        c0 = idx - cbase
        cp_w.wait()
        v00 = plsc.load_gather(win_v, [r0, c0])
        v01 = plsc.load_gather(win_v, [r0, c0 + 1])
        v10 = plsc.load_gather(win_v, [r0 + 1, c0])
        v11 = plsc.load_gather(win_v, [r0 + 1, c0 + 1])
        a0 = v00 + w * (v01 - v00)
        a1 = v10 + w * (v11 - v10)
        out_v[...] = a0 + u * (a1 - a0)
        pltpu.sync_copy(out_v, out_hbm)


def kernel(vols, strikes, expiries, strike, expiry):
    strike_v = jnp.full((L,), strike, dtype=jnp.float32)
    expiry_v = jnp.full((L,), expiry, dtype=jnp.float32)
    mesh = plsc.VectorSubcoreMesh(core_axis_name="c", subcore_axis_name="s")
    run = pl.kernel(
        _body,
        out_type=jax.ShapeDtypeStruct((L,), jnp.float32),
        mesh=mesh,
        compiler_params=pltpu.CompilerParams(needs_layout_passes=False),
        scratch_types=[
            pltpu.VMEM((N_STR,), jnp.float32),
            pltpu.VMEM((N_EXP,), jnp.float32),
            pltpu.VMEM((L,), jnp.float32),
            pltpu.VMEM((L,), jnp.float32),
            pltpu.VMEM((ROWS, COLS), jnp.float32),
            pltpu.VMEM((L,), jnp.float32),
            pltpu.SemaphoreType.DMA,
            pltpu.SemaphoreType.DMA,
            pltpu.SemaphoreType.DMA,
        ],
    )
    out = run(vols, strikes, expiries, strike_v, expiry_v)
    return out[0]


# speedup vs baseline: 5.6314x; 1.0654x over previous
import jax
import jax.numpy as jnp
from jax import lax
from jax.experimental import pallas as pl
from jax.experimental.pallas import tpu as pltpu
from jax.experimental.pallas import tpu_sc as plsc

L = 16
N_EXP = 1024
N_STR = 4096
ROWS = 16
COLS = 256


def _count_le(grid_v, n, strides, x_vec, iota):
    base = iota * 0
    cnt = base
    for st in strides:
        probe = base + iota * st
        in_range = probe < n
        probe_c = jnp.minimum(probe, n - 1)
        vals = plsc.load_gather(grid_v, [probe_c])
        le = jnp.logical_and(vals <= x_vec, in_range)
        cnt = plsc.all_reduce_population_count(le)
        if st != 1:
            base = base + jnp.maximum(cnt - 1, 0) * st
    return base + cnt


def _body(vols_hbm, strikes_hbm, expiries_hbm, strike_hbm, expiry_hbm,
          out_hbm, strikes_v, expiries_v, strike_v, expiry_v, win_v, out_v,
          sem_e, sem_s, sem_w):
    c = lax.axis_index("c")
    s = lax.axis_index("s")

    @pl.when(jnp.logical_and(c == 0, s == 0))
    def _():
        cp_e = pltpu.async_copy(expiries_hbm, expiries_v, sem_e)
        cp_t = pltpu.async_copy(expiry_hbm, expiry_v, sem_e)
        cp_s = pltpu.async_copy(strikes_hbm, strikes_v, sem_s)
        cp_k = pltpu.async_copy(strike_hbm, strike_v, sem_s)

        iota = lax.iota(jnp.int32, L)
        zeros = iota * 0

        cp_e.wait()
        cp_t.wait()
        t = expiry_v[...]
        n_e = _count_le(expiries_v, N_EXP, [64, 4, 1], t, iota)
        jhi = jnp.clip(n_e, 1, N_EXP - 1)
        jlo = jhi - 1
        e_lo = plsc.load_gather(expiries_v, [jlo])
        e_hi = plsc.load_gather(expiries_v, [jhi])
        u = jnp.clip((t - e_lo) / (e_hi - e_lo), 0.0, 1.0)

        cp_s.wait()
        cp_k.wait()
        s_first = plsc.load_gather(strikes_v, [zeros])
        s_last = plsc.load_gather(strikes_v, [zeros + (N_STR - 1)])
        k = jnp.clip(strike_v[...], s_first, s_last)
        n_s = _count_le(strikes_v, N_STR, [256, 16, 1], k, iota)
        idx = jnp.clip(n_s - 1, 0, N_STR - 2)

        rbase = (jlo >> 3) << 3
        cbase = jnp.minimum((idx >> 7) << 7, N_STR - COLS)
        rbase_s = pl.multiple_of(jnp.max(rbase), 8)
        cbase_s = pl.multiple_of(jnp.max(cbase), 128)
        cp_w = pltpu.async_copy(
            vols_hbm.at[pl.ds(rbase_s, ROWS), pl.ds(cbase_s, COLS)],
            win_v, sem_w)

        k_lo = plsc.load_gather(strikes_v, [idx])
        k_hi = plsc.load_gather(strikes_v, [idx + 1])
        w = jnp.clip((k - k_lo) / (k_hi - k_lo), 0.0, 1.0)

        r0 = jlo - rbase
        c0 = idx - cbase
        cp_w.wait()
        v00 = plsc.load_gather(win_v, [r0, c0])
        v01 = plsc.load_gather(win_v, [r0, c0 + 1])
        v10 = plsc.load_gather(win_v, [r0 + 1, c0])
        v11 = plsc.load_gather(win_v, [r0 + 1, c0 + 1])
        a0 = v00 + w * (v01 - v00)
        a1 = v10 + w * (v11 - v10)
        out_v[...] = a0 + u * (a1 - a0)
        pltpu.sync_copy(out_v, out_hbm)


def kernel(vols, strikes, expiries, strike, expiry):
    strike_v = jnp.full((L,), strike, dtype=jnp.float32)
    expiry_v = jnp.full((L,), expiry, dtype=jnp.float32)
    mesh = plsc.VectorSubcoreMesh(core_axis_name="c", subcore_axis_name="s",
                                  num_cores=1)
    run = pl.kernel(
        _body,
        out_type=jax.ShapeDtypeStruct((L,), jnp.float32),
        mesh=mesh,
        compiler_params=pltpu.CompilerParams(needs_layout_passes=False),
        scratch_types=[
            pltpu.VMEM((N_STR,), jnp.float32),
            pltpu.VMEM((N_EXP,), jnp.float32),
            pltpu.VMEM((L,), jnp.float32),
            pltpu.VMEM((L,), jnp.float32),
            pltpu.VMEM((ROWS, COLS), jnp.float32),
            pltpu.VMEM((L,), jnp.float32),
            pltpu.SemaphoreType.DMA,
            pltpu.SemaphoreType.DMA,
            pltpu.SemaphoreType.DMA,
        ],
    )
    out = run(vols, strikes, expiries, strike_v, expiry_v)
    return out[0]


# speedup vs baseline: 5.6551x; 1.0042x over previous
import jax
import jax.numpy as jnp
from jax import lax
from jax.experimental import pallas as pl
from jax.experimental.pallas import tpu as pltpu
from jax.experimental.pallas import tpu_sc as plsc

L = 16
N_EXP = 1024
N_STR = 4096
ROWS = 16
COLS = 256


def _count_le(grid_v, n, strides, x_vec, iota):
    base = iota * 0
    cnt = base
    for st in strides:
        probe = base + iota * st
        in_range = probe < n
        probe_c = jnp.minimum(probe, n - 1)
        vals = plsc.load_gather(grid_v, [probe_c])
        le = jnp.logical_and(vals <= x_vec, in_range)
        cnt = plsc.all_reduce_population_count(le)
        if st != 1:
            base = base + jnp.maximum(cnt - 1, 0) * st
    return base + cnt


def _body(vols_hbm, strikes_hbm, expiries_hbm, strike_hbm, expiry_hbm,
          out_hbm, strikes_v, expiries_v, strike_v, expiry_v, win_v, out_v,
          sem_e, sem_s, sem_w):
    c = lax.axis_index("c")
    s = lax.axis_index("s")

    @pl.when(jnp.logical_and(c == 0, s == 0))
    def _():
        cp_e = pltpu.async_copy(expiries_hbm, expiries_v, sem_e)
        cp_t = pltpu.async_copy(expiry_hbm, expiry_v, sem_e)
        cp_s = pltpu.async_copy(strikes_hbm, strikes_v, sem_s)
        cp_k = pltpu.async_copy(strike_hbm, strike_v, sem_s)

        iota = lax.iota(jnp.int32, L)
        zeros = iota * 0

        cp_e.wait()
        cp_t.wait()
        t = expiry_v[...]
        n_e = _count_le(expiries_v, N_EXP, [64, 4, 1], t, iota)
        jhi = jnp.clip(n_e, 1, N_EXP - 1)
        jlo = jhi - 1
        e_lo = plsc.load_gather(expiries_v, [jlo])
        e_hi = plsc.load_gather(expiries_v, [jhi])
        u = jnp.clip((t - e_lo) / (e_hi - e_lo), 0.0, 1.0)

        cp_s.wait()
        cp_k.wait()
        s_first = plsc.load_gather(strikes_v, [zeros])
        s_last = plsc.load_gather(strikes_v, [zeros + (N_STR - 1)])
        k = jnp.clip(strike_v[...], s_first, s_last)
        n_s = _count_le(strikes_v, N_STR, [256, 16, 1], k, iota)
        idx = jnp.clip(n_s - 1, 0, N_STR - 2)

        rbase = (jlo >> 3) << 3
        cbase = jnp.minimum((idx >> 7) << 7, N_STR - COLS)
        rbase_s = pl.multiple_of(jnp.max(rbase), 8)
        cbase_s = pl.multiple_of(jnp.max(cbase), 128)
        cp_w = pltpu.async_copy(
            vols_hbm.at[pl.ds(rbase_s, ROWS), pl.ds(cbase_s, COLS)],
            win_v, sem_w)

        k_lo = plsc.load_gather(strikes_v, [idx])
        k_hi = plsc.load_gather(strikes_v, [idx + 1])
        w = jnp.clip((k - k_lo) / (k_hi - k_lo), 0.0, 1.0)

        r0 = jlo - rbase
        c0 = idx - cbase
        cp_w.wait()
        v00 = plsc.load_gather(win_v, [r0, c0])
        v01 = plsc.load_gather(win_v, [r0, c0 + 1])
        v10 = plsc.load_gather(win_v, [r0 + 1, c0])
        v11 = plsc.load_gather(win_v, [r0 + 1, c0 + 1])
        a0 = v00 + w * (v01 - v00)
        a1 = v10 + w * (v11 - v10)
        out_v[...] = a0 + u * (a1 - a0)
        pltpu.sync_copy(out_v, out_hbm)


def kernel(vols, strikes, expiries, strike, expiry):
    strike_v = jnp.full((L,), strike, dtype=jnp.float32)
    expiry_v = jnp.full((L,), expiry, dtype=jnp.float32)
    mesh = plsc.VectorSubcoreMesh(core_axis_name="c", subcore_axis_name="s",
                                  num_cores=1, num_subcores=1)
    run = pl.kernel(
        _body,
        out_type=jax.ShapeDtypeStruct((L,), jnp.float32),
        mesh=mesh,
        compiler_params=pltpu.CompilerParams(needs_layout_passes=False),
        scratch_types=[
            pltpu.VMEM((N_STR,), jnp.float32),
            pltpu.VMEM((N_EXP,), jnp.float32),
            pltpu.VMEM((L,), jnp.float32),
            pltpu.VMEM((L,), jnp.float32),
            pltpu.VMEM((ROWS, COLS), jnp.float32),
            pltpu.VMEM((L,), jnp.float32),
            pltpu.SemaphoreType.DMA,
            pltpu.SemaphoreType.DMA,
            pltpu.SemaphoreType.DMA,
        ],
    )
    out = run(vols, strikes, expiries, strike_v, expiry_v)
    return out[0]


# speedup vs baseline: 5.6566x; 1.0003x over previous
import jax
import jax.numpy as jnp
from jax import lax
from jax.experimental import pallas as pl
from jax.experimental.pallas import tpu as pltpu
from jax.experimental.pallas import tpu_sc as plsc

L = 16
N_EXP = 1024
N_STR = 4096
ROWS = 16
COLS = 256


def _count_le(grid_v, n, strides, x_vec, iota):
    base = iota * 0
    cnt = base
    for st in strides:
        probe = base + iota * st
        in_range = probe < n
        probe_c = jnp.minimum(probe, n - 1)
        vals = plsc.load_gather(grid_v, [probe_c])
        le = jnp.logical_and(vals <= x_vec, in_range)
        cnt = plsc.all_reduce_population_count(le)
        if st != 1:
            base = base + jnp.maximum(cnt - 1, 0) * st
    return base + cnt


def _body(vols_hbm, strikes_hbm, expiries_hbm, strike_hbm, expiry_hbm,
          out_hbm, strikes_v, expiries_v, strike_v, expiry_v, win_v, out_v,
          sem_e, sem_s, sem_w):
    c = lax.axis_index("c")
    s = lax.axis_index("s")

    @pl.when(jnp.logical_and(c == 0, s == 0))
    def _():
        cp_e = pltpu.async_copy(expiries_hbm, expiries_v, sem_e)
        cp_t = pltpu.async_copy(expiry_hbm, expiry_v, sem_e)
        cp_s = pltpu.async_copy(strikes_hbm, strikes_v, sem_s)
        cp_k = pltpu.async_copy(strike_hbm, strike_v, sem_s)

        iota = lax.iota(jnp.int32, L)
        zeros = iota * 0

        cp_e.wait()
        cp_t.wait()
        t = plsc.load_gather(expiry_v, [zeros])
        n_e = _count_le(expiries_v, N_EXP, [64, 4, 1], t, iota)
        jhi = jnp.clip(n_e, 1, N_EXP - 1)
        jlo = jhi - 1
        e_lo = plsc.load_gather(expiries_v, [jlo])
        e_hi = plsc.load_gather(expiries_v, [jhi])
        u = jnp.clip((t - e_lo) / (e_hi - e_lo), 0.0, 1.0)

        cp_s.wait()
        cp_k.wait()
        s_first = plsc.load_gather(strikes_v, [zeros])
        s_last = plsc.load_gather(strikes_v, [zeros + (N_STR - 1)])
        k = jnp.clip(plsc.load_gather(strike_v, [zeros]), s_first, s_last)
        n_s = _count_le(strikes_v, N_STR, [256, 16, 1], k, iota)
        idx = jnp.clip(n_s - 1, 0, N_STR - 2)

        rbase = (jlo >> 3) << 3
        cbase = jnp.minimum((idx >> 7) << 7, N_STR - COLS)
        rbase_s = pl.multiple_of(jnp.max(rbase), 8)
        cbase_s = pl.multiple_of(jnp.max(cbase), 128)
        cp_w = pltpu.async_copy(
            vols_hbm.at[pl.ds(rbase_s, ROWS), pl.ds(cbase_s, COLS)],
            win_v, sem_w)

        k_lo = plsc.load_gather(strikes_v, [idx])
        k_hi = plsc.load_gather(strikes_v, [idx + 1])
        w = jnp.clip((k - k_lo) / (k_hi - k_lo), 0.0, 1.0)

        r0 = jlo - rbase
        c0 = idx - cbase
        cp_w.wait()
        v00 = plsc.load_gather(win_v, [r0, c0])
        v01 = plsc.load_gather(win_v, [r0, c0 + 1])
        v10 = plsc.load_gather(win_v, [r0 + 1, c0])
        v11 = plsc.load_gather(win_v, [r0 + 1, c0 + 1])
        a0 = v00 + w * (v01 - v00)
        a1 = v10 + w * (v11 - v10)
        out_v[...] = a0 + u * (a1 - a0)
        pltpu.sync_copy(out_v.at[pl.ds(0, 1)], out_hbm)


def kernel(vols, strikes, expiries, strike, expiry):
    strike_v = strike.astype(jnp.float32).reshape(1)
    expiry_v = expiry.astype(jnp.float32).reshape(1)
    mesh = plsc.VectorSubcoreMesh(core_axis_name="c", subcore_axis_name="s",
                                  num_cores=1, num_subcores=1)
    run = pl.kernel(
        _body,
        out_type=jax.ShapeDtypeStruct((1,), jnp.float32),
        mesh=mesh,
        compiler_params=pltpu.CompilerParams(needs_layout_passes=False),
        scratch_types=[
            pltpu.VMEM((N_STR,), jnp.float32),
            pltpu.VMEM((N_EXP,), jnp.float32),
            pltpu.VMEM((1,), jnp.float32),
            pltpu.VMEM((1,), jnp.float32),
            pltpu.VMEM((ROWS, COLS), jnp.float32),
            pltpu.VMEM((L,), jnp.float32),
            pltpu.SemaphoreType.DMA,
            pltpu.SemaphoreType.DMA,
            pltpu.SemaphoreType.DMA,
        ],
    )
    out = run(vols, strikes, expiries, strike_v, expiry_v)
    return out.reshape(())
